# baseline (speedup 1.0000x reference)
import jax
import jax.numpy as jnp
from jax.experimental import pallas as pl
from jax.experimental.pallas import tpu as pltpu


def _gcn_kernel(f_ref, w_ref, s_ref, o_ref, p_ref):
    @pl.when(pl.program_id(0) == 0)
    def _():
        p_ref[...] = jnp.dot(
            f_ref[...], w_ref[...], preferred_element_type=jnp.float32
        )

    o_ref[...] = jnp.maximum(
        jnp.dot(s_ref[...], p_ref[...], preferred_element_type=jnp.float32),
        0.0,
    )


def kernel(features, support, weight0):
    n, d_in = features.shape
    d_out = weight0.shape[1]
    bm = 200 if n % 200 == 0 else 8
    grid = (n // bm,)
    return pl.pallas_call(
        _gcn_kernel,
        grid=grid,
        in_specs=[
            pl.BlockSpec((n, d_in), lambda i: (0, 0)),
            pl.BlockSpec((d_in, d_out), lambda i: (0, 0)),
            pl.BlockSpec((bm, n), lambda i: (i, 0)),
        ],
        out_specs=pl.BlockSpec((bm, d_out), lambda i: (i, 0)),
        out_shape=jax.ShapeDtypeStruct((n, d_out), jnp.float32),
        scratch_shapes=[pltpu.VMEM((n, d_out), jnp.float32)],
    )(features, weight0, support)


# speedup vs baseline: 1.0060x; 1.0060x over previous
import jax
import jax.numpy as jnp
from jax.experimental import pallas as pl
from jax.experimental.pallas import tpu as pltpu


def _gcn_kernel(f_ref, w_ref, s_ref, o_ref):
    agg = jnp.dot(s_ref[...], f_ref[...], preferred_element_type=jnp.float32)
    o_ref[...] = jnp.maximum(
        jnp.dot(agg, w_ref[...], preferred_element_type=jnp.float32), 0.0
    )


def kernel(features, support, weight0):
    n, d_in = features.shape
    d_out = weight0.shape[1]
    bm = 400 if n % 400 == 0 else 8
    grid = (n // bm,)
    return pl.pallas_call(
        _gcn_kernel,
        grid=grid,
        in_specs=[
            pl.BlockSpec((n, d_in), lambda i: (0, 0)),
            pl.BlockSpec((d_in, d_out), lambda i: (0, 0)),
            pl.BlockSpec((bm, n), lambda i: (i, 0)),
        ],
        out_specs=pl.BlockSpec((bm, d_out), lambda i: (i, 0)),
        out_shape=jax.ShapeDtypeStruct((n, d_out), jnp.float32),
    )(features, weight0, support)
